# baseline (speedup 1.0000x reference)
import functools

import jax
import jax.numpy as jnp
from jax import lax
from jax.experimental import pallas as pl
from jax.experimental.pallas import tpu as pltpu
from jax.experimental.pallas import tpu_sc as plsc

N = 50000
D = 128
H = 128
C = 16
E = 160000
R = 3

NP = 50176
CHUNK = 512
NCHUNK = NP // CHUNK
NS = 16
NC = 2
TSLICE = NP // NS
EPT = E // NS
B = 80
NB = EPT // B
G = 4
GW = 32


def _sc_mesh():
    return plsc.VectorSubcoreMesh(core_axis_name="c", subcore_axis_name="s")


_SC_PARAMS = pltpu.CompilerParams(use_tc_tiling_on_sc=False)


def _deg_body(e0, e1, e2, z1, deg_out, acc, idxb, ones_v, sem_s):
    cid = lax.axis_index("c")
    t = lax.axis_index("s")
    es = (e0, e1, e2)

    for k in range(B // 16):
        ones_v[pl.ds(k * 16, 16)] = jnp.ones((16,), jnp.float32)

    for tau in range(6):
        @pl.when(cid == tau // 3)
        def _():
            pltpu.sync_copy(es[tau // 2].at[tau % 2, pl.ds(t * NB, NB)], idxb)
            pltpu.sync_copy(z1.at[pl.ds(t * TSLICE, TSLICE)],
                            acc.at[pl.ds(t * TSLICE, TSLICE)])
            plsc.subcore_barrier()

            def body(b, _):
                pltpu.async_copy(ones_v, acc.at[idxb.at[b]], sem_s, add=True)
                return _
            lax.fori_loop(0, NB, body, None)

            def drain(b, _):
                pltpu.make_async_copy(z1.at[pl.ds(0, B)], ones_v, sem_s).wait()
                return _
            lax.fori_loop(0, NB, drain, None)
            plsc.subcore_barrier()
            pltpu.sync_copy(acc.at[pl.ds(t * TSLICE, TSLICE)],
                            deg_out.at[tau, pl.ds(t * TSLICE, TSLICE)])
            plsc.subcore_barrier()


def _sc_degrees(es, z1):
    f = pl.kernel(
        _deg_body,
        out_type=jax.ShapeDtypeStruct((6, NP), jnp.float32),
        mesh=_sc_mesh(),
        compiler_params=_SC_PARAMS,
        scratch_types=[
            pltpu.VMEM_SHARED((NP,), jnp.float32),
            pltpu.VMEM((NB, B), jnp.int32),
            pltpu.VMEM((B,), jnp.float32),
            pltpu.SemaphoreType.DMA,
        ],
    )
    return f(es[0], es[1], es[2], z1)


def _prep_body(feat_ref, deg_ref, xs0_ref, xs1_ref, xs2_ref, norms_ref):
    f = feat_ref[...]
    dg = deg_ref[...]
    nrm = lax.rsqrt(jnp.maximum(dg, 1.0))
    norms_ref[...] = nrm
    outs = (xs0_ref, xs1_ref, xs2_ref)
    for r in range(R):
        ns = nrm[r, 0, 0, :]
        outs[r][...] = (f * ns[:, None]).astype(jnp.bfloat16)


def _tc_prep(featp, deg6):
    deg = deg6.reshape(3, 2, 1, NP)
    out_shapes = [jax.ShapeDtypeStruct((NP, D), jnp.bfloat16) for _ in range(R)]
    out_shapes.append(jax.ShapeDtypeStruct((3, 2, 1, NP), jnp.float32))
    xs_spec = pl.BlockSpec((CHUNK, D), lambda i: (i, 0))
    nrm_spec = pl.BlockSpec((3, 2, 1, CHUNK), lambda i: (0, 0, 0, i))
    return pl.pallas_call(
        _prep_body,
        grid=(NCHUNK,),
        in_specs=[xs_spec, nrm_spec],
        out_specs=[xs_spec, xs_spec, xs_spec, nrm_spec],
        out_shape=out_shapes,
    )(featp, deg)


NBUF = 5
OUTER = NB // NBUF


def _agg_body(e0, e1, e2, xv0, xv1, xv2, nd0, nd1, nd2,
              z2, z1, agg_out, q_out,
              acc, accq, sidxb, didxb,
              rw0, rw1, rw2, rw3, rw4, vl0, vl1, vl2, vl3, vl4,
              sg0, sg1, sg2, sg3, sg4, ss0, ss1, ss2, ss3, ss4, semd):
    cid = lax.axis_index("c")
    t = lax.axis_index("s")
    es = (e0, e1, e2)
    xvs = (xv0, xv1, xv2)
    nds = (nd0, nd1, nd2)
    rows = (rw0, rw1, rw2, rw3, rw4)
    vals = (vl0, vl1, vl2, vl3, vl4)
    semg = (sg0, sg1, sg2, sg3, sg4)
    sems = (ss0, ss1, ss2, ss3, ss4)
    rsl = pl.ds(t * TSLICE, TSLICE)

    for c in range(NC):
        @pl.when(cid == c)
        def _():
            pltpu.async_copy(acc.at[rsl],
                             agg_out.at[0, rsl, pl.ds(c * GW, GW)], semd)

    for u in range(R * G):
        r, g = u // G, u % G

        @pl.when(cid == u % 2)
        def _():
            pltpu.sync_copy(es[r].at[0, pl.ds(t * NB, NB)], sidxb)
            pltpu.sync_copy(es[r].at[1, pl.ds(t * NB, NBUF)], didxb)

            def tbody(b, _):
                for k in range(B // 16):
                    sl = pl.ds(k * 16, 16)
                    sidxb[b, sl] = sidxb[b, sl] * G + g
                return _
            lax.fori_loop(0, NB, tbody, None)

            pltpu.make_async_copy(
                acc.at[rsl], agg_out.at[r, rsl, pl.ds(g * GW, GW)],
                semd).wait()
            pltpu.sync_copy(z2.at[rsl], acc.at[rsl])
            plsc.subcore_barrier()

            for j in range(NBUF):
                pltpu.sync_copy(z2.at[pl.ds(0, B)], rows[j])
                pltpu.async_copy(rows[j], acc.at[didxb.at[0]], sems[j],
                                 add=True)

            def obody(o, _):
                descs = []
                for j in range(NBUF):
                    pltpu.make_async_copy(z2.at[pl.ds(0, B)], rows[j],
                                          sems[j]).wait()
                    descs.append(pltpu.async_copy(
                        xvs[r].at[sidxb.at[o * NBUF + j]], rows[j], semg[j]))
                pltpu.sync_copy(es[r].at[1, pl.ds(t * NB + o * NBUF, NBUF)],
                                didxb)
                for j in range(NBUF):
                    descs[j].wait()
                    pltpu.async_copy(rows[j], acc.at[didxb.at[j]],
                                     sems[j], add=True)
                return _
            lax.fori_loop(0, OUTER, obody, None)

            for j in range(NBUF):
                pltpu.make_async_copy(z2.at[pl.ds(0, B)], rows[j],
                                      sems[j]).wait()
            plsc.subcore_barrier()
            pltpu.async_copy(acc.at[rsl],
                             agg_out.at[r, rsl, pl.ds(g * GW, GW)], semd)

    pltpu.make_async_copy(acc.at[rsl],
                          agg_out.at[0, rsl, pl.ds(0, GW)], semd).wait()

    for r in range(R):
        @pl.when(cid == (0 if r == 0 else 1))
        def _():
            pltpu.sync_copy(es[r].at[0, pl.ds(t * NB, NB)], sidxb)
            pltpu.sync_copy(z1.at[pl.ds(t * TSLICE, TSLICE)],
                            accq.at[pl.ds(t * TSLICE, TSLICE)])
            plsc.subcore_barrier()

            for j in range(NBUF):
                pltpu.sync_copy(z1.at[pl.ds(0, B)], vals[j])
                pltpu.async_copy(vals[j], accq.at[sidxb.at[0]], sems[j],
                                 add=True)

            def qbody(o, _):
                pltpu.sync_copy(es[r].at[1, pl.ds(t * NB + o * NBUF, NBUF)],
                                didxb)
                descs = []
                for j in range(NBUF):
                    pltpu.make_async_copy(z1.at[pl.ds(0, B)], vals[j],
                                          sems[j]).wait()
                    descs.append(pltpu.async_copy(
                        nds[r].at[didxb.at[j]], vals[j], semg[j]))
                for j in range(NBUF):
                    descs[j].wait()
                    pltpu.async_copy(vals[j], accq.at[sidxb.at[o * NBUF + j]],
                                     sems[j], add=True)
                return _
            lax.fori_loop(0, OUTER, qbody, None)

            for j in range(NBUF):
                pltpu.make_async_copy(z1.at[pl.ds(0, B)], vals[j],
                                      sems[j]).wait()
            plsc.subcore_barrier()
            pltpu.sync_copy(accq.at[pl.ds(t * TSLICE, TSLICE)],
                            q_out.at[r, pl.ds(t * TSLICE, TSLICE)])
            plsc.subcore_barrier()


def _sc_aggregate(es, xvs, nds, z2, z1):
    f = pl.kernel(
        _agg_body,
        out_type=(
            jax.ShapeDtypeStruct((R, NP, D), jnp.bfloat16),
            jax.ShapeDtypeStruct((R, NP), jnp.float32),
        ),
        mesh=_sc_mesh(),
        compiler_params=_SC_PARAMS,
        scratch_types=(
            [pltpu.VMEM_SHARED((NP, GW), jnp.bfloat16),
             pltpu.VMEM_SHARED((NP,), jnp.float32),
             pltpu.VMEM((NB, B), jnp.int32),
             pltpu.VMEM((NBUF, B), jnp.int32)]
            + [pltpu.VMEM((B, GW), jnp.bfloat16) for _ in range(NBUF)]
            + [pltpu.VMEM((B,), jnp.float32) for _ in range(NBUF)]
            + [pltpu.SemaphoreType.DMA for _ in range(2 * NBUF + 1)]
        ),
    )
    return f(es[0], es[1], es[2],
             xvs[0], xvs[1], xvs[2], nds[0], nds[1], nds[2], z2, z1)


def _finish_body(agg_ref, norms_ref, q_ref, w1_ref, b1_ref, w2_ref, b2_ref,
                 wc_ref, bc_ref, out_ref, s_acc):
    i = pl.program_id(0)

    @pl.when(i == 0)
    def _():
        s_acc[...] = jnp.zeros_like(s_acc)

    b1s = jnp.sum(b1_ref[0], axis=0)
    h = jnp.zeros((CHUNK, H), jnp.float32)
    for r in range(R):
        nd = norms_ref[r, 1, 0, :]
        a = agg_ref[r].astype(jnp.float32) * nd[:, None]
        h = h + jnp.dot(a, w1_ref[r], preferred_element_type=jnp.float32)
    h = jnp.maximum(h + b1s[None, :], 0.0)

    for r in range(R):
        c = norms_ref[r, 0, 0, :] * q_ref[r, 0, :]
        sr = jnp.dot(c[None, :], h, preferred_element_type=jnp.float32)
        s_acc[pl.ds(r, 1), :] = s_acc[pl.ds(r, 1), :] + sr

    @pl.when(i == NCHUNK - 1)
    def _():
        hg = jnp.zeros((1, H), jnp.float32)
        for r in range(R):
            hg = hg + jnp.dot(s_acc[pl.ds(r, 1), :], w2_ref[r],
                              preferred_element_type=jnp.float32)
        b2s = jnp.sum(b2_ref[0], axis=0)
        hg = hg + float(N) * b2s[None, :]
        out_ref[...] = jnp.dot(hg, wc_ref[...],
                               preferred_element_type=jnp.float32) + bc_ref[...]


def _tc_finish(agg, norms, q, W1, b1, W2, b2, Wc, bc):
    return pl.pallas_call(
        _finish_body,
        grid=(NCHUNK,),
        in_specs=[
            pl.BlockSpec((R, CHUNK, D), lambda i: (0, i, 0)),
            pl.BlockSpec((3, 2, 1, CHUNK), lambda i: (0, 0, 0, i)),
            pl.BlockSpec((R, 1, CHUNK), lambda i: (0, 0, i)),
            pl.BlockSpec((R, D, H), lambda i: (0, 0, 0)),
            pl.BlockSpec((1, R, H), lambda i: (0, 0, 0)),
            pl.BlockSpec((R, H, H), lambda i: (0, 0, 0)),
            pl.BlockSpec((1, R, H), lambda i: (0, 0, 0)),
            pl.BlockSpec((H, C), lambda i: (0, 0)),
            pl.BlockSpec((1, C), lambda i: (0, 0)),
        ],
        out_specs=pl.BlockSpec((1, C), lambda i: (0, 0)),
        out_shape=jax.ShapeDtypeStruct((1, C), jnp.float32),
        scratch_shapes=[pltpu.VMEM((8, H), jnp.float32)],
    )(agg, norms, q, W1, b1.reshape(1, R, H),
      W2, b2.reshape(1, R, H), Wc, bc.reshape(1, C))


@jax.jit
def kernel(feat, edge_index_follows, edge_index_likes, edge_index_owns,
           W1, b1, W2, b2, Wc, bc):
    eis = (edge_index_follows, edge_index_likes, edge_index_owns)
    es = [e.reshape(2, E // B, B) for e in eis]
    z1 = jnp.zeros((NP,), jnp.float32)
    z2 = jnp.zeros((NP, GW), jnp.bfloat16)

    deg6 = _sc_degrees(es, z1)
    xs0, xs1, xs2, norms = _tc_prep(feat, deg6)
    xvs = [x.reshape(G * NP, GW) for x in (xs0, xs1, xs2)]
    nds = [norms[r, 1, 0] for r in range(R)]
    agg, q = _sc_aggregate(es, xvs, nds, z2, z1)
    return _tc_finish(agg, norms, q.reshape(R, 1, NP),
                      W1, b1, W2, b2, Wc, bc)


# speedup vs baseline: 1.3778x; 1.3778x over previous
import functools

import jax
import jax.numpy as jnp
from jax import lax
from jax.experimental import pallas as pl
from jax.experimental.pallas import tpu as pltpu
from jax.experimental.pallas import tpu_sc as plsc

N = 50000
D = 128
H = 128
C = 16
E = 160000
R = 3

NP = 50176
CHUNK = 512
NCHUNK = NP // CHUNK
NS = 16
NC = 2
TSLICE = NP // NS
EPT = E // NS
B = 80
NB = EPT // B
G = 4
GW = 32


def _sc_mesh():
    return plsc.VectorSubcoreMesh(core_axis_name="c", subcore_axis_name="s")


_SC_PARAMS = pltpu.CompilerParams(use_tc_tiling_on_sc=False)


def _deg_body(e0, e1, e2, z1, deg_out, acc, idxb, ones_v, sem_s):
    cid = lax.axis_index("c")
    t = lax.axis_index("s")
    es = (e0, e1, e2)

    for k in range(B // 16):
        ones_v[pl.ds(k * 16, 16)] = jnp.ones((16,), jnp.float32)

    for tau in range(6):
        @pl.when(cid == tau // 3)
        def _():
            pltpu.sync_copy(es[tau // 2].at[tau % 2, pl.ds(t * NB, NB)], idxb)
            pltpu.sync_copy(z1.at[pl.ds(t * TSLICE, TSLICE)],
                            acc.at[pl.ds(t * TSLICE, TSLICE)])
            plsc.subcore_barrier()

            def body(b, _):
                pltpu.async_copy(ones_v, acc.at[idxb.at[b]], sem_s, add=True)
                return _
            lax.fori_loop(0, NB, body, None)

            def drain(b, _):
                pltpu.make_async_copy(z1.at[pl.ds(0, B)], ones_v, sem_s).wait()
                return _
            lax.fori_loop(0, NB, drain, None)
            plsc.subcore_barrier()
            pltpu.sync_copy(acc.at[pl.ds(t * TSLICE, TSLICE)],
                            deg_out.at[tau, pl.ds(t * TSLICE, TSLICE)])
            plsc.subcore_barrier()


def _sc_degrees(es, z1):
    f = pl.kernel(
        _deg_body,
        out_type=jax.ShapeDtypeStruct((6, NP), jnp.float32),
        mesh=_sc_mesh(),
        compiler_params=_SC_PARAMS,
        scratch_types=[
            pltpu.VMEM_SHARED((NP,), jnp.float32),
            pltpu.VMEM((NB, B), jnp.int32),
            pltpu.VMEM((B,), jnp.float32),
            pltpu.SemaphoreType.DMA,
        ],
    )
    return f(es[0], es[1], es[2], z1)


def _prep_body(feat_ref, deg_ref, xs0_ref, xs1_ref, xs2_ref, norms_ref):
    f = feat_ref[...]
    dg = deg_ref[...]
    nrm = lax.rsqrt(jnp.maximum(dg, 1.0))
    norms_ref[...] = nrm
    outs = (xs0_ref, xs1_ref, xs2_ref)
    for r in range(R):
        ns = nrm[r, 0, 0, :]
        outs[r][...] = f * ns[:, None]


def _tc_prep(featp, deg6):
    deg = deg6.reshape(3, 2, 1, NP)
    out_shapes = [jax.ShapeDtypeStruct((NP, D), jnp.float32) for _ in range(R)]
    out_shapes.append(jax.ShapeDtypeStruct((3, 2, 1, NP), jnp.float32))
    xs_spec = pl.BlockSpec((CHUNK, D), lambda i: (i, 0))
    nrm_spec = pl.BlockSpec((3, 2, 1, CHUNK), lambda i: (0, 0, 0, i))
    return pl.pallas_call(
        _prep_body,
        grid=(NCHUNK,),
        in_specs=[xs_spec, nrm_spec],
        out_specs=[xs_spec, xs_spec, xs_spec, nrm_spec],
        out_shape=out_shapes,
    )(featp, deg)


NBUF = 5
OUTER = NB // NBUF


def _agg_body(e0, e1, e2, xv0, xv1, xv2, nd0, nd1, nd2,
              z2, z1, agg_out, q_out,
              acc, accq, sidxb, didxb,
              rw0, rw1, rw2, rw3, rw4, vl0, vl1, vl2, vl3, vl4,
              sg0, sg1, sg2, sg3, sg4, ss0, ss1, ss2, ss3, ss4, semd):
    cid = lax.axis_index("c")
    t = lax.axis_index("s")
    es = (e0, e1, e2)
    xvs = (xv0, xv1, xv2)
    nds = (nd0, nd1, nd2)
    rows = (rw0, rw1, rw2, rw3, rw4)
    vals = (vl0, vl1, vl2, vl3, vl4)
    semg = (sg0, sg1, sg2, sg3, sg4)
    sems = (ss0, ss1, ss2, ss3, ss4)
    rsl = pl.ds(t * TSLICE, TSLICE)

    for c in range(NC):
        @pl.when(cid == c)
        def _():
            pltpu.async_copy(acc.at[rsl],
                             agg_out.at[0, rsl, pl.ds(c * GW, GW)], semd)

    for u in range(R * G):
        r, g = u // G, u % G

        @pl.when(cid == u % 2)
        def _():
            pltpu.sync_copy(es[r].at[0, pl.ds(t * NB, NB)], sidxb)
            pltpu.sync_copy(es[r].at[1, pl.ds(t * NB, NBUF)], didxb)

            def tbody(b, _):
                for k in range(B // 16):
                    sl = pl.ds(k * 16, 16)
                    sidxb[b, sl] = sidxb[b, sl] * G + g
                return _
            lax.fori_loop(0, NB, tbody, None)

            pltpu.make_async_copy(
                acc.at[rsl], agg_out.at[r, rsl, pl.ds(g * GW, GW)],
                semd).wait()
            pltpu.sync_copy(z2.at[rsl], acc.at[rsl])
            plsc.subcore_barrier()

            for j in range(NBUF):
                pltpu.sync_copy(z2.at[pl.ds(0, B)], rows[j])
                pltpu.async_copy(rows[j], acc.at[didxb.at[0]], sems[j],
                                 add=True)

            def obody(o, _):
                descs = []
                for j in range(NBUF):
                    pltpu.make_async_copy(z2.at[pl.ds(0, B)], rows[j],
                                          sems[j]).wait()
                    descs.append(pltpu.async_copy(
                        xvs[r].at[sidxb.at[o * NBUF + j]], rows[j], semg[j]))
                pltpu.sync_copy(es[r].at[1, pl.ds(t * NB + o * NBUF, NBUF)],
                                didxb)
                for j in range(NBUF):
                    descs[j].wait()
                    pltpu.async_copy(rows[j], acc.at[didxb.at[j]],
                                     sems[j], add=True)
                return _
            lax.fori_loop(0, OUTER, obody, None)

            for j in range(NBUF):
                pltpu.make_async_copy(z2.at[pl.ds(0, B)], rows[j],
                                      sems[j]).wait()
            plsc.subcore_barrier()
            pltpu.async_copy(acc.at[rsl],
                             agg_out.at[r, rsl, pl.ds(g * GW, GW)], semd)

    pltpu.make_async_copy(acc.at[rsl],
                          agg_out.at[0, rsl, pl.ds(0, GW)], semd).wait()

    for r in range(R):
        @pl.when(cid == (0 if r == 0 else 1))
        def _():
            pltpu.sync_copy(es[r].at[0, pl.ds(t * NB, NB)], sidxb)
            pltpu.sync_copy(z1.at[pl.ds(t * TSLICE, TSLICE)],
                            accq.at[pl.ds(t * TSLICE, TSLICE)])
            plsc.subcore_barrier()

            for j in range(NBUF):
                pltpu.sync_copy(z1.at[pl.ds(0, B)], vals[j])
                pltpu.async_copy(vals[j], accq.at[sidxb.at[0]], sems[j],
                                 add=True)

            def qbody(o, _):
                pltpu.sync_copy(es[r].at[1, pl.ds(t * NB + o * NBUF, NBUF)],
                                didxb)
                descs = []
                for j in range(NBUF):
                    pltpu.make_async_copy(z1.at[pl.ds(0, B)], vals[j],
                                          sems[j]).wait()
                    descs.append(pltpu.async_copy(
                        nds[r].at[didxb.at[j]], vals[j], semg[j]))
                for j in range(NBUF):
                    descs[j].wait()
                    pltpu.async_copy(vals[j], accq.at[sidxb.at[o * NBUF + j]],
                                     sems[j], add=True)
                return _
            lax.fori_loop(0, OUTER, qbody, None)

            for j in range(NBUF):
                pltpu.make_async_copy(z1.at[pl.ds(0, B)], vals[j],
                                      sems[j]).wait()
            plsc.subcore_barrier()
            pltpu.sync_copy(accq.at[pl.ds(t * TSLICE, TSLICE)],
                            q_out.at[r, pl.ds(t * TSLICE, TSLICE)])
            plsc.subcore_barrier()


def _sc_aggregate(es, xvs, nds, z2, z1):
    f = pl.kernel(
        _agg_body,
        out_type=(
            jax.ShapeDtypeStruct((R, NP, D), jnp.float32),
            jax.ShapeDtypeStruct((R, NP), jnp.float32),
        ),
        mesh=_sc_mesh(),
        compiler_params=_SC_PARAMS,
        scratch_types=(
            [pltpu.VMEM_SHARED((NP, GW), jnp.float32),
             pltpu.VMEM_SHARED((NP,), jnp.float32),
             pltpu.VMEM((NB, B), jnp.int32),
             pltpu.VMEM((NBUF, B), jnp.int32)]
            + [pltpu.VMEM((B, GW), jnp.float32) for _ in range(NBUF)]
            + [pltpu.VMEM((B,), jnp.float32) for _ in range(NBUF)]
            + [pltpu.SemaphoreType.DMA for _ in range(2 * NBUF + 1)]
        ),
    )
    return f(es[0], es[1], es[2],
             xvs[0], xvs[1], xvs[2], nds[0], nds[1], nds[2], z2, z1)


def _finish_body(agg_ref, norms_ref, q_ref, w1_ref, b1_ref, w2_ref, b2_ref,
                 wc_ref, bc_ref, out_ref, s_acc):
    i = pl.program_id(0)

    @pl.when(i == 0)
    def _():
        s_acc[...] = jnp.zeros_like(s_acc)

    b1s = jnp.sum(b1_ref[0], axis=0)
    h = jnp.zeros((CHUNK, H), jnp.float32)
    for r in range(R):
        nd = norms_ref[r, 1, 0, :]
        a = agg_ref[r] * nd[:, None]
        h = h + jnp.dot(a, w1_ref[r], preferred_element_type=jnp.float32)
    h = jnp.maximum(h + b1s[None, :], 0.0)

    for r in range(R):
        c = norms_ref[r, 0, 0, :] * q_ref[r, 0, :]
        sr = jnp.dot(c[None, :], h, preferred_element_type=jnp.float32)
        s_acc[pl.ds(r, 1), :] = s_acc[pl.ds(r, 1), :] + sr

    @pl.when(i == NCHUNK - 1)
    def _():
        hg = jnp.zeros((1, H), jnp.float32)
        for r in range(R):
            hg = hg + jnp.dot(s_acc[pl.ds(r, 1), :], w2_ref[r],
                              preferred_element_type=jnp.float32)
        b2s = jnp.sum(b2_ref[0], axis=0)
        hg = hg + float(N) * b2s[None, :]
        out_ref[...] = jnp.dot(hg, wc_ref[...],
                               preferred_element_type=jnp.float32) + bc_ref[...]


def _tc_finish(agg, norms, q, W1, b1, W2, b2, Wc, bc):
    return pl.pallas_call(
        _finish_body,
        grid=(NCHUNK,),
        in_specs=[
            pl.BlockSpec((R, CHUNK, D), lambda i: (0, i, 0)),
            pl.BlockSpec((3, 2, 1, CHUNK), lambda i: (0, 0, 0, i)),
            pl.BlockSpec((R, 1, CHUNK), lambda i: (0, 0, i)),
            pl.BlockSpec((R, D, H), lambda i: (0, 0, 0)),
            pl.BlockSpec((1, R, H), lambda i: (0, 0, 0)),
            pl.BlockSpec((R, H, H), lambda i: (0, 0, 0)),
            pl.BlockSpec((1, R, H), lambda i: (0, 0, 0)),
            pl.BlockSpec((H, C), lambda i: (0, 0)),
            pl.BlockSpec((1, C), lambda i: (0, 0)),
        ],
        out_specs=pl.BlockSpec((1, C), lambda i: (0, 0)),
        out_shape=jax.ShapeDtypeStruct((1, C), jnp.float32),
        scratch_shapes=[pltpu.VMEM((8, H), jnp.float32)],
    )(agg, norms, q, W1, b1.reshape(1, R, H),
      W2, b2.reshape(1, R, H), Wc, bc.reshape(1, C))


@jax.jit
def kernel(feat, edge_index_follows, edge_index_likes, edge_index_owns,
           W1, b1, W2, b2, Wc, bc):
    eis = (edge_index_follows, edge_index_likes, edge_index_owns)
    es = [e.reshape(2, E // B, B) for e in eis]
    z1 = jnp.zeros((NP,), jnp.float32)
    z2 = jnp.zeros((NP, GW), jnp.float32)

    deg6 = _sc_degrees(es, z1)
    xs0, xs1, xs2, norms = _tc_prep(feat, deg6)
    xvs = [x.reshape(G * NP, GW) for x in (xs0, xs1, xs2)]
    nds = [norms[r, 1, 0] for r in range(R)]
    agg, q = _sc_aggregate(es, xvs, nds, z2, z1)
    return _tc_finish(agg, norms, q.reshape(R, 1, NP),
                      W1, b1, W2, b2, Wc, bc)
